# baseline (speedup 1.0000x reference)
import functools

import jax
import jax.numpy as jnp
from jax import lax
from jax.experimental import pallas as pl
from jax.experimental.pallas import tpu as pltpu
from jax.experimental.pallas import tpu_sc as plsc

NC = 2
NS = 16
CH = 128


def _mm2_body(x_ref, w_ref, o_ref):
    o_ref[0] = jnp.dot(x_ref[...], w_ref[0], preferred_element_type=jnp.float32)


def _mm2(x, wstack):
    n, d = x.shape
    h = wstack.shape[2]
    return pl.pallas_call(
        _mm2_body,
        grid=(2,),
        in_specs=[pl.BlockSpec((n, d), lambda c: (0, 0)),
                  pl.BlockSpec((1, d, h), lambda c: (c, 0, 0))],
        out_specs=pl.BlockSpec((1, n, h), lambda c: (c, 0, 0)),
        out_shape=jax.ShapeDtypeStruct((2, n, h), jnp.float32),
    )(x, wstack)


def _bn(h, g, be):
    m = jnp.mean(h, axis=0)
    v = jnp.mean((h - m) ** 2, axis=0)
    return (h - m) * lax.rsqrt(v + 1e-5) * g + be


def _bnmm_body(a_ref, b_ref, g_ref, be_ref, w_ref, o_ref):
    h = _bn(a_ref[0] + b_ref[0, 0], g_ref[0, 0], be_ref[0, 0])
    o_ref[0] = jnp.dot(jnp.maximum(h, 0.0), w_ref[0],
                       preferred_element_type=jnp.float32)


def _bnmm(agg, b, g, be, wstack):
    _, n, h = agg.shape
    return pl.pallas_call(
        _bnmm_body,
        grid=(2,),
        in_specs=[pl.BlockSpec((1, n, h), lambda c: (c, 0, 0)),
                  pl.BlockSpec((1, 1, h), lambda c: (c, 0, 0)),
                  pl.BlockSpec((1, 1, h), lambda c: (c, 0, 0)),
                  pl.BlockSpec((1, 1, h), lambda c: (c, 0, 0)),
                  pl.BlockSpec((1, h, h), lambda c: (c, 0, 0))],
        out_specs=pl.BlockSpec((1, n, h), lambda c: (c, 0, 0)),
        out_shape=jax.ShapeDtypeStruct((2, n, h), jnp.float32),
    )(agg, b, g, be, wstack)


def _final_body(a_ref, b_ref, g_ref, be_ref, wc_ref, bc_ref, o_ref):
    s0 = _bn(a_ref[0] + b_ref[0, 0], g_ref[0, 0], be_ref[0, 0])
    s1 = _bn(a_ref[1] + b_ref[1, 0], g_ref[1, 0], be_ref[1, 0])
    o_ref[...] = (jnp.dot(s0 + s1, wc_ref[...],
                          preferred_element_type=jnp.float32) + bc_ref[0])


def _final(agg, b, g, be, wc, bc):
    _, n, h = agg.shape
    out = wc.shape[1]
    return pl.pallas_call(
        _final_body,
        out_shape=jax.ShapeDtypeStruct((n, out), jnp.float32),
    )(agg, b, g, be, wc, bc.reshape(1, out))


_GDN = lax.GatherDimensionNumbers(
    offset_dims=(), collapsed_slice_dims=(0,), start_index_map=(0,))


def _bcast_lane(wv, l):
    idx = jnp.full((16, 1), l, jnp.int32)
    return lax.gather(wv, idx, _GDN, slice_sizes=(1,),
                      mode=lax.GatherScatterMode.PROMISE_IN_BOUNDS)

def _make_gconv(n, h, nch):
    hh2 = h // 2
    q = (n // (8 * NS)) * 8
    r = n - NS * q
    assert 0 <= r <= CH and r % 8 == 0 and nch % 3 == 0
    kq, kr = divmod(q, CH)
    nlan = hh2 // 16
    nt = nch // 3
    mesh = plsc.VectorSubcoreMesh(core_axis_name="c", subcore_axis_name="s")

    def body(hh0, hh1, src3, dst3, ew3, outf0, outf1, acc, sidx, didx, eww,
             rows, gs0, gs1, gs2, ss0, ss1, ss2):
        cid = lax.axis_index("c")
        sid = lax.axis_index("s")
        tid = cid * NS + sid
        gsem = (gs0, gs1, gs2)
        ssem = (ss0, ss1, ss2)
        row0 = sid * q

        pltpu.sync_copy(src3.at[tid], sidx)
        pltpu.sync_copy(dst3.at[tid], didx)
        pltpu.sync_copy(ew3.at[tid], eww)

        for hh, outf in ((hh0, outf0), (hh1, outf1)):
            @pl.loop(0, CH)
            def _(i):
                for j in range(nlan):
                    rows[0, i, pl.ds(j * 16, 16)] = jnp.zeros(
                        (16,), jnp.float32)

            for k in range(kq):
                pltpu.sync_copy(rows.at[0], acc.at[pl.ds(row0 + k * CH, CH)])
            if kr:
                pltpu.sync_copy(rows.at[0, pl.ds(0, kr)],
                                acc.at[pl.ds(row0 + kq * CH, kr)])
            if r:
                @pl.when(sid == 0)
                def _():
                    pltpu.sync_copy(rows.at[0, pl.ds(0, r)],
                                    acc.at[pl.ds(NS * q, r)])
            plsc.subcore_barrier()

            pltpu.async_copy(hh.at[sidx.at[0]], rows.at[0], gs0)
            pltpu.async_copy(hh.at[sidx.at[1]], rows.at[1], gs1)

            @pl.loop(0, nt)
            def _(t):
                for b in range(3):
                    g = 3 * t + b
                    pltpu.make_async_copy(
                        hh.at[sidx.at[g]], rows.at[b], gsem[b]).wait()

                    @pl.loop(0, CH // 16)
                    def _(gg):
                        wv = eww[g, pl.ds(gg * 16, 16)]
                        for l in range(16):
                            w = _bcast_lane(wv, l)
                            e = gg * 16 + l
                            for j in range(nlan):
                                rows[b, e, pl.ds(j * 16, 16)] = (
                                    rows[b, e, pl.ds(j * 16, 16)] * w)

                    pltpu.async_copy(
                        rows.at[b], acc.at[didx.at[g]], ssem[b], add=True)

                    b1 = (b + 2) % 3
                    if b == 0:
                        @pl.when(t > 0)
                        def _():
                            pltpu.make_async_copy(
                                rows.at[b1], acc.at[didx.at[g - 1]],
                                ssem[b1]).wait()
                    else:
                        pltpu.make_async_copy(
                            rows.at[b1], acc.at[didx.at[g - 1]],
                            ssem[b1]).wait()

                    @pl.when(g + 2 < nch)
                    def _():
                        pltpu.async_copy(
                            hh.at[sidx.at[g + 2]], rows.at[b1], gsem[b1])

            pltpu.make_async_copy(
                rows.at[2], acc.at[didx.at[nch - 1]], ss2).wait()
            plsc.subcore_barrier()

            out0 = cid * n + row0
            for k in range(kq):
                pltpu.sync_copy(acc.at[pl.ds(row0 + k * CH, CH)], rows.at[0])
                pltpu.sync_copy(rows.at[0], outf.at[pl.ds(out0 + k * CH, CH)])
            if kr:
                pltpu.sync_copy(acc.at[pl.ds(row0 + kq * CH, kr)],
                                rows.at[0, pl.ds(0, kr)])
                pltpu.sync_copy(rows.at[0, pl.ds(0, kr)],
                                outf.at[pl.ds(out0 + kq * CH, kr)])
            if r:
                @pl.when(sid == NS - 1)
                def _():
                    pltpu.sync_copy(acc.at[pl.ds(NS * q, r)],
                                    rows.at[0, pl.ds(0, r)])
                    pltpu.sync_copy(rows.at[0, pl.ds(0, r)],
                                    outf.at[pl.ds(cid * n + NS * q, r)])
            plsc.subcore_barrier()

    half = jax.ShapeDtypeStruct((2 * n, hh2), jnp.float32)
    return pl.kernel(
        body,
        out_type=(half, half),
        mesh=mesh,
        compiler_params=pltpu.CompilerParams(use_tc_tiling_on_sc=False),
        scratch_types=[
            pltpu.VMEM_SHARED((n, hh2), jnp.float32),
            pltpu.VMEM((nch, CH), jnp.int32),
            pltpu.VMEM((nch, CH), jnp.int32),
            pltpu.VMEM((nch, CH), jnp.float32),
            pltpu.VMEM((3, CH, hh2), jnp.float32),
            pltpu.SemaphoreType.DMA,
            pltpu.SemaphoreType.DMA,
            pltpu.SemaphoreType.DMA,
            pltpu.SemaphoreType.DMA,
            pltpu.SemaphoreType.DMA,
            pltpu.SemaphoreType.DMA,
        ],
    )


def kernel(x, edge_index, edge_weight, diff_edge, diff_weight, params):
    n, d = x.shape
    h = params['a_W1'].shape[1]
    e = edge_weight.shape[0]

    nch = -(-e // (NS * CH))
    nch += (-nch) % 3
    ep = nch * CH * NS
    padn = ep - e

    def pad(a):
        return jnp.pad(a, (0, padn)).reshape(NS, nch, CH)

    srcf = jnp.concatenate([pad(edge_index[0]), pad(diff_edge[0]) + n])
    dstf = jnp.concatenate([pad(edge_index[1]), pad(diff_edge[1])])
    ewf = jnp.concatenate([pad(edge_weight), pad(diff_weight)])

    p = params
    W1 = jnp.stack([p['a_W1'], p['d_W1']])
    b1 = jnp.stack([p['a_b1'], p['d_b1']]).reshape(2, 1, h)
    g1 = jnp.stack([p['a_g1'], p['d_g1']]).reshape(2, 1, h)
    be1 = jnp.stack([p['a_be1'], p['d_be1']]).reshape(2, 1, h)
    W2 = jnp.stack([p['a_W2'], p['d_W2']])
    b2 = jnp.stack([p['a_b2'], p['d_b2']]).reshape(2, 1, h)
    g2 = jnp.stack([p['a_g2'], p['d_g2']]).reshape(2, 1, h)
    be2 = jnp.stack([p['a_be2'], p['d_be2']]).reshape(2, 1, h)

    gconv = _make_gconv(n, h, nch)
    hf = h // 2

    def conv(hh):
        hhf = hh.reshape(2 * n, h)
        o0, o1 = gconv(hhf[:, :hf], hhf[:, hf:], srcf, dstf, ewf)
        return jnp.concatenate(
            [o0.reshape(2, n, hf), o1.reshape(2, n, hf)], axis=2)

    hh1 = _mm2(x, W1)
    agg1 = conv(hh1)
    hh2 = _bnmm(agg1, b1, g1, be1, W2)
    agg2 = conv(hh2)
    return _final(agg2, b2, g2, be2, p['Wc'], p['bc'])


# speedup vs baseline: 1.3480x; 1.3480x over previous
import functools

import jax
import jax.numpy as jnp
from jax import lax
from jax.experimental import pallas as pl
from jax.experimental.pallas import tpu as pltpu
from jax.experimental.pallas import tpu_sc as plsc

NC = 2
NS = 16
CH = 128


def _mm2_body(x_ref, w_ref, o_ref):
    o_ref[0] = jnp.dot(x_ref[...], w_ref[0], preferred_element_type=jnp.float32)


def _mm2(x, wstack):
    n, d = x.shape
    h = wstack.shape[2]
    return pl.pallas_call(
        _mm2_body,
        grid=(2,),
        in_specs=[pl.BlockSpec((n, d), lambda c: (0, 0)),
                  pl.BlockSpec((1, d, h), lambda c: (c, 0, 0))],
        out_specs=pl.BlockSpec((1, n, h), lambda c: (c, 0, 0)),
        out_shape=jax.ShapeDtypeStruct((2, n, h), jnp.float32),
    )(x, wstack)


def _bn(h, g, be):
    m = jnp.mean(h, axis=0)
    v = jnp.mean((h - m) ** 2, axis=0)
    return (h - m) * lax.rsqrt(v + 1e-5) * g + be


def _bnmm_body(a_ref, b_ref, g_ref, be_ref, w_ref, o_ref):
    h = _bn(a_ref[0] + b_ref[0, 0], g_ref[0, 0], be_ref[0, 0])
    o_ref[0] = jnp.dot(jnp.maximum(h, 0.0), w_ref[0],
                       preferred_element_type=jnp.float32)


def _bnmm(agg, b, g, be, wstack):
    _, n, h = agg.shape
    return pl.pallas_call(
        _bnmm_body,
        grid=(2,),
        in_specs=[pl.BlockSpec((1, n, h), lambda c: (c, 0, 0)),
                  pl.BlockSpec((1, 1, h), lambda c: (c, 0, 0)),
                  pl.BlockSpec((1, 1, h), lambda c: (c, 0, 0)),
                  pl.BlockSpec((1, 1, h), lambda c: (c, 0, 0)),
                  pl.BlockSpec((1, h, h), lambda c: (c, 0, 0))],
        out_specs=pl.BlockSpec((1, n, h), lambda c: (c, 0, 0)),
        out_shape=jax.ShapeDtypeStruct((2, n, h), jnp.float32),
    )(agg, b, g, be, wstack)


def _final_body(a_ref, b_ref, g_ref, be_ref, wc_ref, bc_ref, o_ref):
    s0 = _bn(a_ref[0] + b_ref[0, 0], g_ref[0, 0], be_ref[0, 0])
    s1 = _bn(a_ref[1] + b_ref[1, 0], g_ref[1, 0], be_ref[1, 0])
    o_ref[...] = (jnp.dot(s0 + s1, wc_ref[...],
                          preferred_element_type=jnp.float32) + bc_ref[0])


def _final(agg, b, g, be, wc, bc):
    _, n, h = agg.shape
    out = wc.shape[1]
    return pl.pallas_call(
        _final_body,
        out_shape=jax.ShapeDtypeStruct((n, out), jnp.float32),
    )(agg, b, g, be, wc, bc.reshape(1, out))


_GDN = lax.GatherDimensionNumbers(
    offset_dims=(), collapsed_slice_dims=(0,), start_index_map=(0,))


def _bcast_lane(wv, l):
    idx = jnp.full((16, 1), l, jnp.int32)
    return lax.gather(wv, idx, _GDN, slice_sizes=(1,),
                      mode=lax.GatherScatterMode.PROMISE_IN_BOUNDS)

def _make_gconv(n, h, nch):
    hh2 = h // 2
    q = (n // (8 * NS)) * 8
    r = n - NS * q
    assert 0 <= r <= CH and r % 8 == 0 and nch % 3 == 0
    kq, kr = divmod(q, CH)
    nlan = hh2 // 16
    nt = nch // 3
    mesh = plsc.VectorSubcoreMesh(core_axis_name="c", subcore_axis_name="s")

    def body(hh0, hh1, src3, dst3, ew3, outf0, outf1, acc, sidx, didx, eww,
             rows, gs0, gs1, gs2, ss0, ss1, ss2):
        cid = lax.axis_index("c")
        sid = lax.axis_index("s")
        tid = cid * NS + sid
        gsem = (gs0, gs1, gs2)
        ssem = (ss0, ss1, ss2)
        row0 = sid * q

        pltpu.sync_copy(src3.at[tid], sidx)
        pltpu.sync_copy(dst3.at[tid], didx)
        pltpu.sync_copy(ew3.at[tid], eww)

        for hh, outf in ((hh0, outf0), (hh1, outf1)):
            @pl.loop(0, CH)
            def _(i):
                for j in range(nlan):
                    rows[0, i, pl.ds(j * 16, 16)] = jnp.zeros(
                        (16,), jnp.float32)

            for k in range(kq):
                pltpu.sync_copy(rows.at[0], acc.at[pl.ds(row0 + k * CH, CH)])
            if kr:
                pltpu.sync_copy(rows.at[0, pl.ds(0, kr)],
                                acc.at[pl.ds(row0 + kq * CH, kr)])
            if r:
                @pl.when(sid == 0)
                def _():
                    pltpu.sync_copy(rows.at[0, pl.ds(0, r)],
                                    acc.at[pl.ds(NS * q, r)])
            plsc.subcore_barrier()

            pltpu.async_copy(hh.at[sidx.at[0]], rows.at[0], gs0)
            pltpu.async_copy(hh.at[sidx.at[1]], rows.at[1], gs1)

            @pl.loop(0, nt)
            def _(t):
                for b in range(3):
                    g = 3 * t + b
                    pltpu.make_async_copy(
                        hh.at[sidx.at[g]], rows.at[b], gsem[b]).wait()

                    @plsc.parallel_loop(0, CH // 16, 1, unroll=2)
                    def _(gg):
                        wv = eww[g, pl.ds(gg * 16, 16)]
                        for l in range(16):
                            w = _bcast_lane(wv, l)
                            e = gg * 16 + l
                            vals = [rows[b, e, pl.ds(j * 16, 16)] * w
                                    for j in range(nlan)]
                            for j in range(nlan):
                                rows[b, e, pl.ds(j * 16, 16)] = vals[j]

                    pltpu.async_copy(
                        rows.at[b], acc.at[didx.at[g]], ssem[b], add=True)

                    b1 = (b + 2) % 3
                    if b == 0:
                        @pl.when(t > 0)
                        def _():
                            pltpu.make_async_copy(
                                rows.at[b1], acc.at[didx.at[g - 1]],
                                ssem[b1]).wait()
                    else:
                        pltpu.make_async_copy(
                            rows.at[b1], acc.at[didx.at[g - 1]],
                            ssem[b1]).wait()

                    @pl.when(g + 2 < nch)
                    def _():
                        pltpu.async_copy(
                            hh.at[sidx.at[g + 2]], rows.at[b1], gsem[b1])

            pltpu.make_async_copy(
                rows.at[2], acc.at[didx.at[nch - 1]], ss2).wait()
            plsc.subcore_barrier()

            out0 = cid * n + row0
            for k in range(kq):
                pltpu.sync_copy(acc.at[pl.ds(row0 + k * CH, CH)], rows.at[0])
                pltpu.sync_copy(rows.at[0], outf.at[pl.ds(out0 + k * CH, CH)])
            if kr:
                pltpu.sync_copy(acc.at[pl.ds(row0 + kq * CH, kr)],
                                rows.at[0, pl.ds(0, kr)])
                pltpu.sync_copy(rows.at[0, pl.ds(0, kr)],
                                outf.at[pl.ds(out0 + kq * CH, kr)])
            if r:
                @pl.when(sid == NS - 1)
                def _():
                    pltpu.sync_copy(acc.at[pl.ds(NS * q, r)],
                                    rows.at[0, pl.ds(0, r)])
                    pltpu.sync_copy(rows.at[0, pl.ds(0, r)],
                                    outf.at[pl.ds(cid * n + NS * q, r)])
            plsc.subcore_barrier()

    half = jax.ShapeDtypeStruct((2 * n, hh2), jnp.float32)
    return pl.kernel(
        body,
        out_type=(half, half),
        mesh=mesh,
        compiler_params=pltpu.CompilerParams(use_tc_tiling_on_sc=False),
        scratch_types=[
            pltpu.VMEM_SHARED((n, hh2), jnp.float32),
            pltpu.VMEM((nch, CH), jnp.int32),
            pltpu.VMEM((nch, CH), jnp.int32),
            pltpu.VMEM((nch, CH), jnp.float32),
            pltpu.VMEM((3, CH, hh2), jnp.float32),
            pltpu.SemaphoreType.DMA,
            pltpu.SemaphoreType.DMA,
            pltpu.SemaphoreType.DMA,
            pltpu.SemaphoreType.DMA,
            pltpu.SemaphoreType.DMA,
            pltpu.SemaphoreType.DMA,
        ],
    )


def kernel(x, edge_index, edge_weight, diff_edge, diff_weight, params):
    n, d = x.shape
    h = params['a_W1'].shape[1]
    e = edge_weight.shape[0]

    nch = -(-e // (NS * CH))
    nch += (-nch) % 3
    ep = nch * CH * NS
    padn = ep - e

    def pad(a):
        return jnp.pad(a, (0, padn)).reshape(NS, nch, CH)

    srcf = jnp.concatenate([pad(edge_index[0]), pad(diff_edge[0]) + n])
    dstf = jnp.concatenate([pad(edge_index[1]), pad(diff_edge[1])])
    ewf = jnp.concatenate([pad(edge_weight), pad(diff_weight)])

    p = params
    W1 = jnp.stack([p['a_W1'], p['d_W1']])
    b1 = jnp.stack([p['a_b1'], p['d_b1']]).reshape(2, 1, h)
    g1 = jnp.stack([p['a_g1'], p['d_g1']]).reshape(2, 1, h)
    be1 = jnp.stack([p['a_be1'], p['d_be1']]).reshape(2, 1, h)
    W2 = jnp.stack([p['a_W2'], p['d_W2']])
    b2 = jnp.stack([p['a_b2'], p['d_b2']]).reshape(2, 1, h)
    g2 = jnp.stack([p['a_g2'], p['d_g2']]).reshape(2, 1, h)
    be2 = jnp.stack([p['a_be2'], p['d_be2']]).reshape(2, 1, h)

    gconv = _make_gconv(n, h, nch)
    hf = h // 2

    def conv(hh):
        hhf = hh.reshape(2 * n, h)
        o0, o1 = gconv(hhf[:, :hf], hhf[:, hf:], srcf, dstf, ewf)
        return jnp.concatenate(
            [o0.reshape(2, n, hf), o1.reshape(2, n, hf)], axis=2)

    hh1 = _mm2(x, W1)
    agg1 = conv(hh1)
    hh2 = _bnmm(agg1, b1, g1, be1, W2)
    agg2 = conv(hh2)
    return _final(agg2, b2, g2, be2, p['Wc'], p['bc'])


# speedup vs baseline: 2.1354x; 1.5841x over previous
import functools

import jax
import jax.numpy as jnp
from jax import lax
from jax.experimental import pallas as pl
from jax.experimental.pallas import tpu as pltpu
from jax.experimental.pallas import tpu_sc as plsc

NC = 2
NS = 16
CH = 128


def _mm2_body(x_ref, w_ref, o0_ref, o1_ref):
    hf = w_ref.shape[2] // 2
    res = jnp.dot(x_ref[...], w_ref[0], preferred_element_type=jnp.float32)
    o0_ref[0] = res[:, :hf]
    o1_ref[0] = res[:, hf:]


def _mm2(x, wstack):
    n, d = x.shape
    h = wstack.shape[2]
    hf = h // 2
    half = jax.ShapeDtypeStruct((2, n, hf), jnp.float32)
    return pl.pallas_call(
        _mm2_body,
        grid=(2,),
        in_specs=[pl.BlockSpec((n, d), lambda c: (0, 0)),
                  pl.BlockSpec((1, d, h), lambda c: (c, 0, 0))],
        out_specs=[pl.BlockSpec((1, n, hf), lambda c: (c, 0, 0)),
                   pl.BlockSpec((1, n, hf), lambda c: (c, 0, 0))],
        out_shape=(half, half),
    )(x, wstack)


def _bn(h, g, be):
    m = jnp.mean(h, axis=0)
    v = jnp.mean((h - m) ** 2, axis=0)
    return (h - m) * lax.rsqrt(v + 1e-5) * g + be


def _bnmm_body(a0_ref, a1_ref, b_ref, g_ref, be_ref, w_ref, o0_ref, o1_ref):
    hf = a0_ref.shape[2]
    h0 = _bn(a0_ref[0] + b_ref[0, 0, :hf], g_ref[0, 0, :hf],
             be_ref[0, 0, :hf])
    h1 = _bn(a1_ref[0] + b_ref[0, 0, hf:], g_ref[0, 0, hf:],
             be_ref[0, 0, hf:])
    h0 = jnp.maximum(h0, 0.0)
    h1 = jnp.maximum(h1, 0.0)
    w = w_ref[0]
    o0_ref[0] = (jnp.dot(h0, w[:hf, :hf], preferred_element_type=jnp.float32)
                 + jnp.dot(h1, w[hf:, :hf],
                           preferred_element_type=jnp.float32))
    o1_ref[0] = (jnp.dot(h0, w[:hf, hf:], preferred_element_type=jnp.float32)
                 + jnp.dot(h1, w[hf:, hf:],
                           preferred_element_type=jnp.float32))


def _bnmm(a0, a1, b, g, be, wstack):
    _, n, hf = a0.shape
    h = 2 * hf
    half = jax.ShapeDtypeStruct((2, n, hf), jnp.float32)
    spec = pl.BlockSpec((1, n, hf), lambda c: (c, 0, 0))
    pspec = pl.BlockSpec((1, 1, h), lambda c: (c, 0, 0))
    return pl.pallas_call(
        _bnmm_body,
        grid=(2,),
        in_specs=[spec, spec, pspec, pspec, pspec,
                  pl.BlockSpec((1, h, h), lambda c: (c, 0, 0))],
        out_specs=[spec, spec],
        out_shape=(half, half),
    )(a0, a1, b, g, be, wstack)


def _final_body(a0_ref, a1_ref, b_ref, g_ref, be_ref, wc_ref, bc_ref, o_ref):
    hf = a0_ref.shape[2]
    s0 = (_bn(a0_ref[0] + b_ref[0, 0, :hf], g_ref[0, 0, :hf],
              be_ref[0, 0, :hf])
          + _bn(a0_ref[1] + b_ref[1, 0, :hf], g_ref[1, 0, :hf],
                be_ref[1, 0, :hf]))
    s1 = (_bn(a1_ref[0] + b_ref[0, 0, hf:], g_ref[0, 0, hf:],
              be_ref[0, 0, hf:])
          + _bn(a1_ref[1] + b_ref[1, 0, hf:], g_ref[1, 0, hf:],
                be_ref[1, 0, hf:]))
    wc = wc_ref[...]
    o_ref[...] = (jnp.dot(s0, wc[:hf], preferred_element_type=jnp.float32)
                  + jnp.dot(s1, wc[hf:], preferred_element_type=jnp.float32)
                  + bc_ref[0])


def _final(a0, a1, b, g, be, wc, bc):
    _, n, hf = a0.shape
    out = wc.shape[1]
    return pl.pallas_call(
        _final_body,
        out_shape=jax.ShapeDtypeStruct((n, out), jnp.float32),
    )(a0, a1, b, g, be, wc, bc.reshape(1, out))


_GDN = lax.GatherDimensionNumbers(
    offset_dims=(), collapsed_slice_dims=(0,), start_index_map=(0,))


def _bcast_lane(wv, l):
    idx = jnp.full((16, 1), l, jnp.int32)
    return lax.gather(wv, idx, _GDN, slice_sizes=(1,),
                      mode=lax.GatherScatterMode.PROMISE_IN_BOUNDS)

def _make_gconv(n, h, nch):
    hh2 = h // 2
    q = (n // (8 * NS)) * 8
    r = n - NS * q
    assert 0 <= r <= CH and r % 8 == 0 and nch % 4 == 0
    kq, kr = divmod(q, CH)
    nlan = hh2 // 16
    nt = nch // 4
    mesh = plsc.VectorSubcoreMesh(core_axis_name="c", subcore_axis_name="s")

    def body(hh0, hh1, ed3, outf0, outf1, acc, tbl, sdw, rows,
             gs0, gs1, gs2, gs3, ss0, ss1, ss2, ss3, is0, is1, is2, is3):
        cid = lax.axis_index("c")
        sid = lax.axis_index("s")
        tid = cid * NS + sid
        gsem = (gs0, gs1, gs2, gs3)
        ssem = (ss0, ss1, ss2, ss3)
        isem = (is0, is1, is2, is3)
        row0 = sid * q

        def gather_desc(j, slot, sem):
            return pltpu.make_async_copy(
                tbl.at[sdw.at[j % 8, 0]], rows.at[slot], sem)

        def scatter_start(j, slot, sem):
            pltpu.async_copy(
                rows.at[slot], acc.at[sdw.at[j % 8, 1]], sem, add=True)

        def scatter_wait(j, slot, sem):
            pltpu.make_async_copy(
                rows.at[slot], acc.at[sdw.at[j % 8, 1]], sem).wait()

        for hh, outf in ((hh0, outf0), (hh1, outf1)):
            tb0 = cid * n + row0
            for k in range(kq):
                pltpu.sync_copy(hh.at[pl.ds(tb0 + k * CH, CH)],
                                tbl.at[pl.ds(row0 + k * CH, CH)])
            if kr:
                pltpu.sync_copy(hh.at[pl.ds(tb0 + kq * CH, kr)],
                                tbl.at[pl.ds(row0 + kq * CH, kr)])
            if r:
                @pl.when(sid == 0)
                def _():
                    pltpu.sync_copy(hh.at[pl.ds(cid * n + NS * q, r)],
                                    tbl.at[pl.ds(NS * q, r)])

            @pl.loop(0, CH)
            def _(i):
                for j in range(nlan):
                    rows[0, i, pl.ds(j * 16, 16)] = jnp.zeros(
                        (16,), jnp.float32)

            for k in range(kq):
                pltpu.sync_copy(rows.at[0], acc.at[pl.ds(row0 + k * CH, CH)])
            if kr:
                pltpu.sync_copy(rows.at[0, pl.ds(0, kr)],
                                acc.at[pl.ds(row0 + kq * CH, kr)])
            if r:
                @pl.when(sid == 0)
                def _():
                    pltpu.sync_copy(rows.at[0, pl.ds(0, r)],
                                    acc.at[pl.ds(NS * q, r)])
            plsc.subcore_barrier()

            for j in range(4):
                pltpu.sync_copy(ed3.at[tid, j], sdw.at[j])
            pltpu.async_copy(tbl.at[sdw.at[0, 0]], rows.at[0], gs0)
            pltpu.async_copy(tbl.at[sdw.at[1, 0]], rows.at[1], gs1)

            @pl.loop(0, nt)
            def _(t):
                for b in range(4):
                    g = 4 * t + b
                    gather_desc(g, b, gsem[b]).wait()

                    @plsc.parallel_loop(0, CH // 16, 1, unroll=4)
                    def _(gg):
                        wv = lax.bitcast_convert_type(
                            sdw[g % 8, 2, pl.ds(gg * 16, 16)], jnp.float32)
                        for l in range(16):
                            w = _bcast_lane(wv, l)
                            e = gg * 16 + l
                            vals = [rows[b, e, pl.ds(j * 16, 16)] * w
                                    for j in range(nlan)]
                            for j in range(nlan):
                                rows[b, e, pl.ds(j * 16, 16)] = vals[j]

                    scatter_start(g, b, ssem[b])

                    b2 = (b + 2) % 4
                    if b < 2:
                        @pl.when(t > 0)
                        def _():
                            scatter_wait(g - 2, b2, ssem[b2])
                    else:
                        scatter_wait(g - 2, b2, ssem[b2])

                    @pl.when(g + 2 < nch)
                    def _():
                        if b < 2:
                            @pl.when(t > 0)
                            def _():
                                pltpu.make_async_copy(
                                    ed3.at[tid, g + 2], sdw.at[(g + 2) % 8],
                                    isem[b2]).wait()
                        else:
                            pltpu.make_async_copy(
                                ed3.at[tid, g + 2], sdw.at[(g + 2) % 8],
                                isem[b2]).wait()
                        gather_desc(g + 2, b2, gsem[b2]).start()

                    @pl.when(g + 4 < nch)
                    def _():
                        pltpu.async_copy(
                            ed3.at[tid, g + 4], sdw.at[(g + 4) % 8], isem[b])

            scatter_wait(nch - 2, 2, ss2)
            scatter_wait(nch - 1, 3, ss3)
            plsc.subcore_barrier()

            out0 = cid * n + row0
            for k in range(kq):
                pltpu.sync_copy(acc.at[pl.ds(row0 + k * CH, CH)], rows.at[0])
                pltpu.sync_copy(rows.at[0], outf.at[pl.ds(out0 + k * CH, CH)])
            if kr:
                pltpu.sync_copy(acc.at[pl.ds(row0 + kq * CH, kr)],
                                rows.at[0, pl.ds(0, kr)])
                pltpu.sync_copy(rows.at[0, pl.ds(0, kr)],
                                outf.at[pl.ds(out0 + kq * CH, kr)])
            if r:
                @pl.when(sid == NS - 1)
                def _():
                    pltpu.sync_copy(acc.at[pl.ds(NS * q, r)],
                                    rows.at[0, pl.ds(0, r)])
                    pltpu.sync_copy(rows.at[0, pl.ds(0, r)],
                                    outf.at[pl.ds(cid * n + NS * q, r)])
            plsc.subcore_barrier()

    half = jax.ShapeDtypeStruct((2 * n, hh2), jnp.float32)
    return pl.kernel(
        body,
        out_type=(half, half),
        mesh=mesh,
        compiler_params=pltpu.CompilerParams(use_tc_tiling_on_sc=False),
        scratch_types=[
            pltpu.VMEM_SHARED((n, hh2), jnp.float32),
            pltpu.VMEM_SHARED((n, hh2), jnp.float32),
            pltpu.VMEM((8, 3, CH), jnp.int32),
            pltpu.VMEM((4, CH, hh2), jnp.float32),
        ] + [pltpu.SemaphoreType.DMA] * 12,
    )


def kernel(x, edge_index, edge_weight, diff_edge, diff_weight, params):
    n, d = x.shape
    h = params['a_W1'].shape[1]
    e = edge_weight.shape[0]

    nch = -(-e // (NS * CH))
    nch += (-nch) % 4
    ep = nch * CH * NS
    padn = ep - e

    def pad(a):
        return jnp.pad(a, (0, padn)).reshape(NS, nch, CH)

    srcf = jnp.concatenate([pad(edge_index[0]), pad(diff_edge[0])])
    dstf = jnp.concatenate([pad(edge_index[1]), pad(diff_edge[1])])
    ewf = jnp.concatenate([pad(edge_weight), pad(diff_weight)])
    ed3 = jnp.stack(
        [srcf, dstf, lax.bitcast_convert_type(ewf, jnp.int32)], axis=2)

    p = params
    W1 = jnp.stack([p['a_W1'], p['d_W1']])
    b1 = jnp.stack([p['a_b1'], p['d_b1']]).reshape(2, 1, h)
    g1 = jnp.stack([p['a_g1'], p['d_g1']]).reshape(2, 1, h)
    be1 = jnp.stack([p['a_be1'], p['d_be1']]).reshape(2, 1, h)
    W2 = jnp.stack([p['a_W2'], p['d_W2']])
    b2 = jnp.stack([p['a_b2'], p['d_b2']]).reshape(2, 1, h)
    g2 = jnp.stack([p['a_g2'], p['d_g2']]).reshape(2, 1, h)
    be2 = jnp.stack([p['a_be2'], p['d_be2']]).reshape(2, 1, h)

    gconv = _make_gconv(n, h, nch)
    hf = h // 2

    def conv(hh0, hh1):
        o0, o1 = gconv(hh0.reshape(2 * n, hf), hh1.reshape(2 * n, hf), ed3)
        return o0.reshape(2, n, hf), o1.reshape(2, n, hf)

    a0, a1 = conv(*_mm2(x, W1))
    h0, h1 = _bnmm(a0, a1, b1, g1, be1, W2)
    a0, a1 = conv(h0, h1)
    return _final(a0, a1, b2, g2, be2, p['Wc'], p['bc'])


# speedup vs baseline: 2.1436x; 1.0039x over previous
import functools

import jax
import jax.numpy as jnp
from jax import lax
from jax.experimental import pallas as pl
from jax.experimental.pallas import tpu as pltpu
from jax.experimental.pallas import tpu_sc as plsc

NC = 2
NS = 16
CH = 128


def _mm2_body(x_ref, w_ref, o0_ref, o1_ref):
    hf = w_ref.shape[2] // 2
    res = jnp.dot(x_ref[...], w_ref[0], preferred_element_type=jnp.float32)
    o0_ref[0] = res[:, :hf]
    o1_ref[0] = res[:, hf:]


def _mm2(x, wstack):
    n, d = x.shape
    h = wstack.shape[2]
    hf = h // 2
    half = jax.ShapeDtypeStruct((2, n, hf), jnp.float32)
    return pl.pallas_call(
        _mm2_body,
        grid=(2,),
        in_specs=[pl.BlockSpec((n, d), lambda c: (0, 0)),
                  pl.BlockSpec((1, d, h), lambda c: (c, 0, 0))],
        out_specs=[pl.BlockSpec((1, n, hf), lambda c: (c, 0, 0)),
                   pl.BlockSpec((1, n, hf), lambda c: (c, 0, 0))],
        out_shape=(half, half),
    )(x, wstack)


def _bn(h, g, be):
    m = jnp.mean(h, axis=0)
    v = jnp.mean((h - m) ** 2, axis=0)
    return (h - m) * lax.rsqrt(v + 1e-5) * g + be


def _bnmm_body(a0_ref, a1_ref, b_ref, g_ref, be_ref, w_ref, o0_ref, o1_ref):
    hf = a0_ref.shape[2]
    h0 = _bn(a0_ref[0] + b_ref[0, 0, :hf], g_ref[0, 0, :hf],
             be_ref[0, 0, :hf])
    h1 = _bn(a1_ref[0] + b_ref[0, 0, hf:], g_ref[0, 0, hf:],
             be_ref[0, 0, hf:])
    h0 = jnp.maximum(h0, 0.0)
    h1 = jnp.maximum(h1, 0.0)
    w = w_ref[0]
    o0_ref[0] = (jnp.dot(h0, w[:hf, :hf], preferred_element_type=jnp.float32)
                 + jnp.dot(h1, w[hf:, :hf],
                           preferred_element_type=jnp.float32))
    o1_ref[0] = (jnp.dot(h0, w[:hf, hf:], preferred_element_type=jnp.float32)
                 + jnp.dot(h1, w[hf:, hf:],
                           preferred_element_type=jnp.float32))


def _bnmm(a0, a1, b, g, be, wstack):
    _, n, hf = a0.shape
    h = 2 * hf
    half = jax.ShapeDtypeStruct((2, n, hf), jnp.float32)
    spec = pl.BlockSpec((1, n, hf), lambda c: (c, 0, 0))
    pspec = pl.BlockSpec((1, 1, h), lambda c: (c, 0, 0))
    return pl.pallas_call(
        _bnmm_body,
        grid=(2,),
        in_specs=[spec, spec, pspec, pspec, pspec,
                  pl.BlockSpec((1, h, h), lambda c: (c, 0, 0))],
        out_specs=[spec, spec],
        out_shape=(half, half),
    )(a0, a1, b, g, be, wstack)


def _final_body(a0_ref, a1_ref, b_ref, g_ref, be_ref, wc_ref, bc_ref, o_ref):
    hf = a0_ref.shape[2]
    s0 = (_bn(a0_ref[0] + b_ref[0, 0, :hf], g_ref[0, 0, :hf],
              be_ref[0, 0, :hf])
          + _bn(a0_ref[1] + b_ref[1, 0, :hf], g_ref[1, 0, :hf],
                be_ref[1, 0, :hf]))
    s1 = (_bn(a1_ref[0] + b_ref[0, 0, hf:], g_ref[0, 0, hf:],
              be_ref[0, 0, hf:])
          + _bn(a1_ref[1] + b_ref[1, 0, hf:], g_ref[1, 0, hf:],
                be_ref[1, 0, hf:]))
    wc = wc_ref[...]
    o_ref[...] = (jnp.dot(s0, wc[:hf], preferred_element_type=jnp.float32)
                  + jnp.dot(s1, wc[hf:], preferred_element_type=jnp.float32)
                  + bc_ref[0])


def _final(a0, a1, b, g, be, wc, bc):
    _, n, hf = a0.shape
    out = wc.shape[1]
    return pl.pallas_call(
        _final_body,
        out_shape=jax.ShapeDtypeStruct((n, out), jnp.float32),
    )(a0, a1, b, g, be, wc, bc.reshape(1, out))


_GDN = lax.GatherDimensionNumbers(
    offset_dims=(), collapsed_slice_dims=(0,), start_index_map=(0,))


def _bcast_lane(wv, l):
    idx = jnp.full((16, 1), l, jnp.int32)
    return lax.gather(wv, idx, _GDN, slice_sizes=(1,),
                      mode=lax.GatherScatterMode.PROMISE_IN_BOUNDS)

def _make_gconv(n, h, nch):
    hh2 = h // 2
    q = (n // (8 * NS)) * 8
    r = n - NS * q
    assert 0 <= r <= CH and r % 8 == 0 and nch % 4 == 0
    kq, kr = divmod(q, CH)
    nlan = hh2 // 16
    nt = nch // 4
    mesh = plsc.VectorSubcoreMesh(core_axis_name="c", subcore_axis_name="s")

    def body(hh0, hh1, ed3, outf0, outf1, acc, tbl, sdw, rows,
             gs0, gs1, gs2, gs3, ss0, ss1, ss2, ss3, is0, is1, is2, is3):
        cid = lax.axis_index("c")
        sid = lax.axis_index("s")
        tid = cid * NS + sid
        gsem = (gs0, gs1, gs2, gs3)
        ssem = (ss0, ss1, ss2, ss3)
        isem = (is0, is1, is2, is3)
        row0 = sid * q

        def gather_desc(j, slot, sem):
            return pltpu.make_async_copy(
                tbl.at[sdw.at[j % 8, 0]], rows.at[slot], sem)

        def scatter_start(j, slot, sem):
            pltpu.async_copy(
                rows.at[slot], acc.at[sdw.at[j % 8, 1]], sem, add=True)

        def scatter_wait(j, slot, sem):
            pltpu.make_async_copy(
                rows.at[slot], acc.at[sdw.at[j % 8, 1]], sem).wait()

        for hh, outf in ((hh0, outf0), (hh1, outf1)):
            tb0 = cid * n + row0
            for k in range(kq):
                pltpu.sync_copy(hh.at[pl.ds(tb0 + k * CH, CH)],
                                tbl.at[pl.ds(row0 + k * CH, CH)])
            if kr:
                pltpu.sync_copy(hh.at[pl.ds(tb0 + kq * CH, kr)],
                                tbl.at[pl.ds(row0 + kq * CH, kr)])
            if r:
                @pl.when(sid == 0)
                def _():
                    pltpu.sync_copy(hh.at[pl.ds(cid * n + NS * q, r)],
                                    tbl.at[pl.ds(NS * q, r)])

            @pl.loop(0, CH)
            def _(i):
                for j in range(nlan):
                    rows[0, i, pl.ds(j * 16, 16)] = jnp.zeros(
                        (16,), jnp.float32)

            for k in range(kq):
                pltpu.sync_copy(rows.at[0], acc.at[pl.ds(row0 + k * CH, CH)])
            if kr:
                pltpu.sync_copy(rows.at[0, pl.ds(0, kr)],
                                acc.at[pl.ds(row0 + kq * CH, kr)])
            if r:
                @pl.when(sid == 0)
                def _():
                    pltpu.sync_copy(rows.at[0, pl.ds(0, r)],
                                    acc.at[pl.ds(NS * q, r)])
            plsc.subcore_barrier()

            for j in range(4):
                pltpu.sync_copy(ed3.at[tid, j], sdw.at[j])
            pltpu.async_copy(tbl.at[sdw.at[0, 0]], rows.at[0], gs0)
            pltpu.async_copy(tbl.at[sdw.at[1, 0]], rows.at[1], gs1)

            @pl.loop(0, nt)
            def _(t):
                for b in range(4):
                    g = 4 * t + b
                    gather_desc(g, b, gsem[b]).wait()

                    @plsc.parallel_loop(0, CH // 16, 1, unroll=2)
                    def _(gg):
                        wv = lax.bitcast_convert_type(
                            sdw[g % 8, 2, pl.ds(gg * 16, 16)], jnp.float32)
                        for l in range(16):
                            w = _bcast_lane(wv, l)
                            e = gg * 16 + l
                            vals = [rows[b, e, pl.ds(j * 16, 16)] * w
                                    for j in range(nlan)]
                            for j in range(nlan):
                                rows[b, e, pl.ds(j * 16, 16)] = vals[j]

                    scatter_start(g, b, ssem[b])

                    b2 = (b + 2) % 4
                    if b < 2:
                        @pl.when(t > 0)
                        def _():
                            scatter_wait(g - 2, b2, ssem[b2])
                    else:
                        scatter_wait(g - 2, b2, ssem[b2])

                    @pl.when(g + 2 < nch)
                    def _():
                        if b < 2:
                            @pl.when(t > 0)
                            def _():
                                pltpu.make_async_copy(
                                    ed3.at[tid, g + 2], sdw.at[(g + 2) % 8],
                                    isem[b2]).wait()
                        else:
                            pltpu.make_async_copy(
                                ed3.at[tid, g + 2], sdw.at[(g + 2) % 8],
                                isem[b2]).wait()
                        gather_desc(g + 2, b2, gsem[b2]).start()

                    @pl.when(g + 4 < nch)
                    def _():
                        pltpu.async_copy(
                            ed3.at[tid, g + 4], sdw.at[(g + 4) % 8], isem[b])

            scatter_wait(nch - 2, 2, ss2)
            scatter_wait(nch - 1, 3, ss3)
            plsc.subcore_barrier()

            out0 = cid * n + row0
            for k in range(kq):
                pltpu.sync_copy(acc.at[pl.ds(row0 + k * CH, CH)], rows.at[0])
                pltpu.sync_copy(rows.at[0], outf.at[pl.ds(out0 + k * CH, CH)])
            if kr:
                pltpu.sync_copy(acc.at[pl.ds(row0 + kq * CH, kr)],
                                rows.at[0, pl.ds(0, kr)])
                pltpu.sync_copy(rows.at[0, pl.ds(0, kr)],
                                outf.at[pl.ds(out0 + kq * CH, kr)])
            if r:
                @pl.when(sid == NS - 1)
                def _():
                    pltpu.sync_copy(acc.at[pl.ds(NS * q, r)],
                                    rows.at[0, pl.ds(0, r)])
                    pltpu.sync_copy(rows.at[0, pl.ds(0, r)],
                                    outf.at[pl.ds(cid * n + NS * q, r)])
            plsc.subcore_barrier()

    half = jax.ShapeDtypeStruct((2 * n, hh2), jnp.float32)
    return pl.kernel(
        body,
        out_type=(half, half),
        mesh=mesh,
        compiler_params=pltpu.CompilerParams(use_tc_tiling_on_sc=False),
        scratch_types=[
            pltpu.VMEM_SHARED((n, hh2), jnp.float32),
            pltpu.VMEM_SHARED((n, hh2), jnp.float32),
            pltpu.VMEM((8, 3, CH), jnp.int32),
            pltpu.VMEM((4, CH, hh2), jnp.float32),
        ] + [pltpu.SemaphoreType.DMA] * 12,
    )


def kernel(x, edge_index, edge_weight, diff_edge, diff_weight, params):
    n, d = x.shape
    h = params['a_W1'].shape[1]
    e = edge_weight.shape[0]

    nch = -(-e // (NS * CH))
    nch += (-nch) % 4
    ep = nch * CH * NS
    padn = ep - e

    def pad(a):
        return jnp.pad(a, (0, padn)).reshape(NS, nch, CH)

    srcf = jnp.concatenate([pad(edge_index[0]), pad(diff_edge[0])])
    dstf = jnp.concatenate([pad(edge_index[1]), pad(diff_edge[1])])
    ewf = jnp.concatenate([pad(edge_weight), pad(diff_weight)])
    ed3 = jnp.stack(
        [srcf, dstf, lax.bitcast_convert_type(ewf, jnp.int32)], axis=2)

    p = params
    W1 = jnp.stack([p['a_W1'], p['d_W1']])
    b1 = jnp.stack([p['a_b1'], p['d_b1']]).reshape(2, 1, h)
    g1 = jnp.stack([p['a_g1'], p['d_g1']]).reshape(2, 1, h)
    be1 = jnp.stack([p['a_be1'], p['d_be1']]).reshape(2, 1, h)
    W2 = jnp.stack([p['a_W2'], p['d_W2']])
    b2 = jnp.stack([p['a_b2'], p['d_b2']]).reshape(2, 1, h)
    g2 = jnp.stack([p['a_g2'], p['d_g2']]).reshape(2, 1, h)
    be2 = jnp.stack([p['a_be2'], p['d_be2']]).reshape(2, 1, h)

    gconv = _make_gconv(n, h, nch)
    hf = h // 2

    def conv(hh0, hh1):
        o0, o1 = gconv(hh0.reshape(2 * n, hf), hh1.reshape(2 * n, hf), ed3)
        return o0.reshape(2, n, hf), o1.reshape(2, n, hf)

    a0, a1 = conv(*_mm2(x, W1))
    h0, h1 = _bnmm(a0, a1, b1, g1, be1, W2)
    a0, a1 = conv(h0, h1)
    return _final(a0, a1, b2, g2, be2, p['Wc'], p['bc'])
